# Initial kernel scaffold; baseline (speedup 1.0000x reference)
import jax
import jax.numpy as jnp
from jax.experimental import pallas as pl


def kernel(x, vision, attention_mask, ln_g, ln_b, Wq, Wkv, Wo):
    raise NotImplementedError("write your pallas kernel here")


# speedup vs baseline: 51.1673x; 51.1673x over previous
import functools

import jax
import jax.numpy as jnp
from jax.experimental import pallas as pl
from jax.experimental.pallas import tpu as pltpu

HEADS = 8
DIM_HEAD = 64
T_TILE = 256


def _fused_kernel(x_ref, m_ref, g_ref, bt_ref, wq_ref, vis_ref, wkv_ref,
                  wo_ref, o_ref, k_scr, vo_scr, *, inner, V):
    t = pl.program_id(1)

    @pl.when(t == 0)
    def _prep():
        vis = vis_ref[0]
        kv = jnp.dot(vis, wkv_ref[...], preferred_element_type=jnp.float32)
        k_scr[...] = kv[:, :inner]
        for h in range(HEADS):
            vh = kv[:, inner + h * DIM_HEAD:inner + (h + 1) * DIM_HEAD]
            wo_h = wo_ref[h * DIM_HEAD:(h + 1) * DIM_HEAD, :]
            vo_scr[h * V:(h + 1) * V, :] = jnp.dot(
                vh, wo_h, preferred_element_type=jnp.float32)

    xb = x_ref[0]
    mu = jnp.mean(xb, axis=-1, keepdims=True)
    var = jnp.mean((xb - mu) ** 2, axis=-1, keepdims=True)
    xn = (xb - mu) * jax.lax.rsqrt(var + 1e-5) * g_ref[0] + bt_ref[0]
    q = jnp.dot(xn, wq_ref[...],
                preferred_element_type=jnp.float32) * (DIM_HEAD ** -0.5)

    mf = (m_ref[0] != 0).astype(jnp.float32)
    neg = (mf - 1.0) * 10000.0
    k = k_scr[...]
    cols = []
    for h in range(HEADS):
        qh = q[:, h * DIM_HEAD:(h + 1) * DIM_HEAD]
        kh = k[:, h * DIM_HEAD:(h + 1) * DIM_HEAD]
        s = jax.lax.dot_general(qh, kh, (((1,), (1,)), ((), ())),
                                preferred_element_type=jnp.float32) + neg
        s = s - jnp.max(s, axis=-1, keepdims=True)
        e = jnp.exp(s)
        p = e / jnp.sum(e, axis=-1, keepdims=True)
        cols.append(p * mf)
    attn = jnp.concatenate(cols, axis=-1)
    o_ref[0] = jnp.dot(attn, vo_scr[...], preferred_element_type=jnp.float32)


def kernel(x, vision, attention_mask, ln_g, ln_b, Wq, Wkv, Wo):
    B, T, C = x.shape
    V = vision.shape[1]
    inner = HEADS * DIM_HEAD
    mask_t = jnp.transpose(attention_mask, (0, 2, 1)).astype(jnp.int32)
    g2 = ln_g.reshape(1, C)
    b2 = ln_b.reshape(1, C)
    grid = (B, T // T_TILE)
    return pl.pallas_call(
        functools.partial(_fused_kernel, inner=inner, V=V),
        grid=grid,
        in_specs=[
            pl.BlockSpec((1, T_TILE, C), lambda b, t: (b, t, 0)),
            pl.BlockSpec((1, T_TILE, V), lambda b, t: (b, t, 0)),
            pl.BlockSpec((1, C), lambda b, t: (0, 0)),
            pl.BlockSpec((1, C), lambda b, t: (0, 0)),
            pl.BlockSpec((C, inner), lambda b, t: (0, 0)),
            pl.BlockSpec((1, V, C), lambda b, t: (b, 0, 0)),
            pl.BlockSpec((C, 2 * inner), lambda b, t: (0, 0)),
            pl.BlockSpec((inner, C), lambda b, t: (0, 0)),
        ],
        out_specs=pl.BlockSpec((1, T_TILE, C), lambda b, t: (b, t, 0)),
        out_shape=jax.ShapeDtypeStruct((B, T, C), jnp.float32),
        scratch_shapes=[
            pltpu.VMEM((V, inner), jnp.float32),
            pltpu.VMEM((HEADS * V, C), jnp.float32),
        ],
    )(x, mask_t, g2, b2, Wq, vision, Wkv, Wo)


# speedup vs baseline: 55.9688x; 1.0938x over previous
import functools

import jax
import jax.numpy as jnp
from jax.experimental import pallas as pl
from jax.experimental.pallas import tpu as pltpu

HEADS = 8
DIM_HEAD = 64
T_TILE = 512


def _fused_kernel(x_ref, m_ref, g_ref, bt_ref, wq_ref, vis_ref, wkv_ref,
                  wo_ref, o_ref, k_scr, vo_scr, *, inner, V):
    t = pl.program_id(1)

    @pl.when(t == 0)
    def _prep():
        vis = vis_ref[0]
        kv = jnp.dot(vis, wkv_ref[...], preferred_element_type=jnp.float32)
        k_scr[...] = kv[:, :inner]
        for h in range(HEADS):
            vh = kv[:, inner + h * DIM_HEAD:inner + (h + 1) * DIM_HEAD]
            wo_h = wo_ref[h * DIM_HEAD:(h + 1) * DIM_HEAD, :]
            vo_scr[h * V:(h + 1) * V, :] = jnp.dot(
                vh, wo_h, preferred_element_type=jnp.float32)

    xb = x_ref[0]
    mu = jnp.mean(xb, axis=-1, keepdims=True)
    var = jnp.mean((xb - mu) ** 2, axis=-1, keepdims=True)
    xn = (xb - mu) * jax.lax.rsqrt(var + 1e-5) * g_ref[0] + bt_ref[0]
    q = jnp.dot(xn, wq_ref[...],
                preferred_element_type=jnp.float32) * (DIM_HEAD ** -0.5)

    mf = (m_ref[0] != 0).astype(jnp.float32)
    neg = (mf - 1.0) * 10000.0
    k = k_scr[...]
    cols = []
    for h in range(HEADS):
        qh = q[:, h * DIM_HEAD:(h + 1) * DIM_HEAD]
        kh = k[:, h * DIM_HEAD:(h + 1) * DIM_HEAD]
        s = jax.lax.dot_general(qh, kh, (((1,), (1,)), ((), ())),
                                preferred_element_type=jnp.float32) + neg
        s = s - jnp.max(s, axis=-1, keepdims=True)
        e = jnp.exp(s)
        p = e / jnp.sum(e, axis=-1, keepdims=True)
        cols.append(p * mf)
    attn = jnp.concatenate(cols, axis=-1)
    o_ref[0] = jnp.dot(attn, vo_scr[...], preferred_element_type=jnp.float32)


def kernel(x, vision, attention_mask, ln_g, ln_b, Wq, Wkv, Wo):
    B, T, C = x.shape
    V = vision.shape[1]
    inner = HEADS * DIM_HEAD
    mask_t = jnp.transpose(attention_mask, (0, 2, 1)).astype(jnp.int32)
    g2 = ln_g.reshape(1, C)
    b2 = ln_b.reshape(1, C)
    grid = (B, T // T_TILE)
    return pl.pallas_call(
        functools.partial(_fused_kernel, inner=inner, V=V),
        grid=grid,
        in_specs=[
            pl.BlockSpec((1, T_TILE, C), lambda b, t: (b, t, 0)),
            pl.BlockSpec((1, T_TILE, V), lambda b, t: (b, t, 0)),
            pl.BlockSpec((1, C), lambda b, t: (0, 0)),
            pl.BlockSpec((1, C), lambda b, t: (0, 0)),
            pl.BlockSpec((C, inner), lambda b, t: (0, 0)),
            pl.BlockSpec((1, V, C), lambda b, t: (b, 0, 0)),
            pl.BlockSpec((C, 2 * inner), lambda b, t: (0, 0)),
            pl.BlockSpec((inner, C), lambda b, t: (0, 0)),
        ],
        out_specs=pl.BlockSpec((1, T_TILE, C), lambda b, t: (b, t, 0)),
        out_shape=jax.ShapeDtypeStruct((B, T, C), jnp.float32),
        scratch_shapes=[
            pltpu.VMEM((V, inner), jnp.float32),
            pltpu.VMEM((HEADS * V, C), jnp.float32),
        ],
    )(x, mask_t, g2, b2, Wq, vision, Wkv, Wo)


# speedup vs baseline: 64.4118x; 1.1509x over previous
import functools

import jax
import jax.numpy as jnp
from jax.experimental import pallas as pl
from jax.experimental.pallas import tpu as pltpu

HEADS = 8
DIM_HEAD = 64
T_TILE = 1024


def _fused_kernel(x_ref, m_ref, g_ref, bt_ref, wq_ref, vis_ref, wkv_ref,
                  wo_ref, o_ref, k_scr, vo_scr, *, inner, V):
    t = pl.program_id(1)

    @pl.when(t == 0)
    def _prep():
        vis = vis_ref[0]
        kv = jnp.dot(vis, wkv_ref[...], preferred_element_type=jnp.float32)
        k_scr[...] = kv[:, :inner]
        for h in range(HEADS):
            vh = kv[:, inner + h * DIM_HEAD:inner + (h + 1) * DIM_HEAD]
            wo_h = wo_ref[h * DIM_HEAD:(h + 1) * DIM_HEAD, :]
            vo_scr[h * V:(h + 1) * V, :] = jnp.dot(
                vh, wo_h, preferred_element_type=jnp.float32)

    xb = x_ref[0]
    mu = jnp.mean(xb, axis=-1, keepdims=True)
    var = jnp.mean((xb - mu) ** 2, axis=-1, keepdims=True)
    xn = (xb - mu) * jax.lax.rsqrt(var + 1e-5) * g_ref[0] + bt_ref[0]
    q = jnp.dot(xn, wq_ref[...],
                preferred_element_type=jnp.float32) * (DIM_HEAD ** -0.5)

    mf = (m_ref[0] != 0).astype(jnp.float32)
    neg = (mf - 1.0) * 10000.0
    k = k_scr[...]
    cols = []
    for h in range(HEADS):
        qh = q[:, h * DIM_HEAD:(h + 1) * DIM_HEAD]
        kh = k[:, h * DIM_HEAD:(h + 1) * DIM_HEAD]
        s = jax.lax.dot_general(qh, kh, (((1,), (1,)), ((), ())),
                                preferred_element_type=jnp.float32) + neg
        s = s - jnp.max(s, axis=-1, keepdims=True)
        e = jnp.exp(s)
        p = e / jnp.sum(e, axis=-1, keepdims=True)
        cols.append(p * mf)
    attn = jnp.concatenate(cols, axis=-1)
    o_ref[0] = jnp.dot(attn, vo_scr[...], preferred_element_type=jnp.float32)


def kernel(x, vision, attention_mask, ln_g, ln_b, Wq, Wkv, Wo):
    B, T, C = x.shape
    V = vision.shape[1]
    inner = HEADS * DIM_HEAD
    mask_t = jnp.transpose(attention_mask, (0, 2, 1)).astype(jnp.int32)
    g2 = ln_g.reshape(1, C)
    b2 = ln_b.reshape(1, C)
    grid = (B, T // T_TILE)
    return pl.pallas_call(
        functools.partial(_fused_kernel, inner=inner, V=V),
        grid=grid,
        in_specs=[
            pl.BlockSpec((1, T_TILE, C), lambda b, t: (b, t, 0)),
            pl.BlockSpec((1, T_TILE, V), lambda b, t: (b, t, 0)),
            pl.BlockSpec((1, C), lambda b, t: (0, 0)),
            pl.BlockSpec((1, C), lambda b, t: (0, 0)),
            pl.BlockSpec((C, inner), lambda b, t: (0, 0)),
            pl.BlockSpec((1, V, C), lambda b, t: (b, 0, 0)),
            pl.BlockSpec((C, 2 * inner), lambda b, t: (0, 0)),
            pl.BlockSpec((inner, C), lambda b, t: (0, 0)),
        ],
        out_specs=pl.BlockSpec((1, T_TILE, C), lambda b, t: (b, t, 0)),
        out_shape=jax.ShapeDtypeStruct((B, T, C), jnp.float32),
        scratch_shapes=[
            pltpu.VMEM((V, inner), jnp.float32),
            pltpu.VMEM((HEADS * V, C), jnp.float32),
        ],
    )(x, mask_t, g2, b2, Wq, vision, Wkv, Wo)


# speedup vs baseline: 107.4894x; 1.6688x over previous
import functools

import jax
import jax.numpy as jnp
from jax.experimental import pallas as pl
from jax.experimental.pallas import tpu as pltpu

HEADS = 8
DIM_HEAD = 64
T_TILE = 1024


def _fused_kernel(x_ref, m_ref, g_ref, bt_ref, wq_ref, vis_ref, wkv_ref,
                  wo_ref, o_ref, kbd_scr, vo_scr, ocol_scr, orow_scr, *,
                  inner, V):
    t = pl.program_id(1)

    @pl.when(t == 0)
    def _prep():
        vis = vis_ref[0]
        kv = jnp.dot(vis, wkv_ref[...], preferred_element_type=jnp.float32)
        scale = jnp.float32(DIM_HEAD ** -0.5)
        kbd_scr[...] = jnp.zeros((inner, HEADS * V), jnp.float32)
        for h in range(HEADS):
            kh = kv[:, h * DIM_HEAD:(h + 1) * DIM_HEAD]
            kbd_scr[h * DIM_HEAD:(h + 1) * DIM_HEAD,
                    h * V:(h + 1) * V] = kh.T * scale
            vh = kv[:, inner + h * DIM_HEAD:inner + (h + 1) * DIM_HEAD]
            wo_h = wo_ref[h * DIM_HEAD:(h + 1) * DIM_HEAD, :]
            vo_scr[h * V:(h + 1) * V, :] = jnp.dot(
                vh, wo_h, preferred_element_type=jnp.float32)
        seg_c = jax.lax.broadcasted_iota(jnp.int32, (HEADS * V, HEADS), 0)
        hd_c = jax.lax.broadcasted_iota(jnp.int32, (HEADS * V, HEADS), 1)
        ocol_scr[...] = (seg_c // V == hd_c).astype(jnp.float32)
        seg_r = jax.lax.broadcasted_iota(jnp.int32, (HEADS, HEADS * V), 1)
        hd_r = jax.lax.broadcasted_iota(jnp.int32, (HEADS, HEADS * V), 0)
        orow_scr[...] = (seg_r // V == hd_r).astype(jnp.float32)

    xb = x_ref[0]
    mu = jnp.mean(xb, axis=-1, keepdims=True)
    var = jnp.mean((xb - mu) ** 2, axis=-1, keepdims=True)
    xn = (xb - mu) * jax.lax.rsqrt(var + 1e-5) * g_ref[0] + bt_ref[0]
    q = jnp.dot(xn, wq_ref[...], preferred_element_type=jnp.float32)

    mt = m_ref[0].T
    neg = jnp.where(mt != 0, 0.0, -10000.0).astype(jnp.float32)
    neg8 = jnp.concatenate([neg] * HEADS, axis=-1)

    sim8 = jnp.dot(q, kbd_scr[...], preferred_element_type=jnp.float32)
    e8 = jnp.exp(sim8 + neg8)
    d8 = jnp.dot(e8, ocol_scr[...], preferred_element_type=jnp.float32)
    r8 = 1.0 / jnp.maximum(d8, 1e-30)
    rfull = jnp.dot(r8, orow_scr[...], preferred_element_type=jnp.float32)
    p = e8 * rfull
    o_ref[0] = jnp.dot(p, vo_scr[...], preferred_element_type=jnp.float32)


def kernel(x, vision, attention_mask, ln_g, ln_b, Wq, Wkv, Wo):
    B, T, C = x.shape
    V = vision.shape[1]
    inner = HEADS * DIM_HEAD
    g2 = ln_g.reshape(1, C)
    b2 = ln_b.reshape(1, C)
    grid = (B, T // T_TILE)
    return pl.pallas_call(
        functools.partial(_fused_kernel, inner=inner, V=V),
        grid=grid,
        in_specs=[
            pl.BlockSpec((1, T_TILE, C), lambda b, t: (b, t, 0)),
            pl.BlockSpec((1, V, T_TILE), lambda b, t: (b, 0, t)),
            pl.BlockSpec((1, C), lambda b, t: (0, 0)),
            pl.BlockSpec((1, C), lambda b, t: (0, 0)),
            pl.BlockSpec((C, inner), lambda b, t: (0, 0)),
            pl.BlockSpec((1, V, C), lambda b, t: (b, 0, 0)),
            pl.BlockSpec((C, 2 * inner), lambda b, t: (0, 0)),
            pl.BlockSpec((inner, C), lambda b, t: (0, 0)),
        ],
        out_specs=pl.BlockSpec((1, T_TILE, C), lambda b, t: (b, t, 0)),
        out_shape=jax.ShapeDtypeStruct((B, T, C), jnp.float32),
        scratch_shapes=[
            pltpu.VMEM((inner, HEADS * V), jnp.float32),
            pltpu.VMEM((HEADS * V, C), jnp.float32),
            pltpu.VMEM((HEADS * V, HEADS), jnp.float32),
            pltpu.VMEM((HEADS, HEADS * V), jnp.float32),
        ],
    )(x, attention_mask.astype(jnp.int32), g2, b2, Wq, vision, Wkv, Wo)
